# Initial kernel scaffold; baseline (speedup 1.0000x reference)
import jax
import jax.numpy as jnp
from jax.experimental import pallas as pl


def kernel(x, Wr, br, gr, betar, W1, b1, g1, bt1, W2, b2, g2, bt2):
    raise NotImplementedError("write your pallas kernel here")


# speedup vs baseline: 1.8706x; 1.8706x over previous
import jax
import jax.numpy as jnp
from jax.experimental import pallas as pl
from jax.experimental.pallas import tpu as pltpu

T, B, C, H, W = 4, 16, 256, 14, 14
HW = H * W
E, K = 8, 2
HID, OUT = 256, 256
_BN_INV = 1.0 / jnp.sqrt(jnp.float32(1.0 + 1e-5))


def _router_body(x_ref, wr_ref, br_ref, idx_ref, wk_ref):
    x = x_ref[:, 0]
    v = jnp.zeros((C, HW), jnp.float32)
    srows = []
    for t in range(T):
        v = v + (x[t] - v) / 2.0
        s = ((v - 1.0) >= 0.0).astype(jnp.float32)
        v = v * (1.0 - s)
        srows.append(jnp.sum(s, axis=-1, keepdims=True))
    S = jnp.concatenate(srows, axis=1)
    logits = jax.lax.dot_general(
        S, wr_ref[...], (((0,), (1,)), ((), ())),
        preferred_element_type=jnp.float32)
    logits = logits * (1.0 / HW) + br_ref[...]
    iota = jax.lax.broadcasted_iota(jnp.int32, (T, E), 1)
    m1 = jnp.max(logits, axis=-1, keepdims=True)
    i1 = jnp.min(jnp.where(logits >= m1, iota, E), axis=-1, keepdims=True)
    l2 = jnp.where(iota == i1, -jnp.inf, logits)
    m2 = jnp.max(l2, axis=-1, keepdims=True)
    i2 = jnp.min(jnp.where(l2 >= m2, iota, E), axis=-1, keepdims=True)
    w1 = 1.0 / (1.0 + jnp.exp(m2 - m1))
    idx_ref[0] = jnp.concatenate([i1, i2], axis=1)
    wk_ref[0] = jnp.concatenate([w1, 1.0 - w1], axis=1)


def _expert_body(idx_ref, wk_ref, taus_ref, tok_ref, w1_ref, b1_ref,
                 w2_ref, b2_ref, out_ref):
    n = pl.program_id(0)
    tok = tok_ref[0]
    acc = jnp.zeros((OUT, HW), jnp.float32)
    for k in range(K):
        e = idx_ref[n, k]
        tau = taus_ref[e]
        s1 = ((tok / tau - 1.0) >= 0.0).astype(jnp.float32)
        h = jnp.dot(w1_ref[e], s1, preferred_element_type=jnp.float32) + b1_ref[e]
        x2 = tok + h
        s2 = ((x2 / tau - 1.0) >= 0.0).astype(jnp.float32)
        o = jnp.dot(w2_ref[e], s2, preferred_element_type=jnp.float32) + b2_ref[e]
        acc = acc + wk_ref[n, k] * (o + x2)
    out_ref[0] = acc


def kernel(x, Wr, br, gr, betar, W1, b1, g1, bt1, W2, b2, g2, bt2):
    f32 = jnp.float32
    x2d = x.reshape(T, B, C, HW)
    taus = jnp.linspace(1.5, 4.0, E).astype(f32)
    ar = gr * _BN_INV
    wr_eff = Wr * ar[:, None]
    br_eff = (br * ar + betar).reshape(1, E)
    a1 = g1 * _BN_INV
    w1_eff = W1 * a1[:, :, None]
    b1_eff = (b1 * a1 + bt1).reshape(E, HID, 1)
    a2 = g2 * _BN_INV
    w2_eff = W2 * a2[:, :, None]
    b2_eff = (b2 * a2 + bt2).reshape(E, OUT, 1)

    idx_bt, wk_bt = pl.pallas_call(
        _router_body,
        grid=(B,),
        in_specs=[
            pl.BlockSpec((T, 1, C, HW), lambda b: (0, b, 0, 0)),
            pl.BlockSpec((E, C), lambda b: (0, 0)),
            pl.BlockSpec((1, E), lambda b: (0, 0)),
        ],
        out_specs=[
            pl.BlockSpec((1, T, K), lambda b: (b, 0, 0)),
            pl.BlockSpec((1, T, K), lambda b: (b, 0, 0)),
        ],
        out_shape=[
            jax.ShapeDtypeStruct((B, T, K), jnp.int32),
            jax.ShapeDtypeStruct((B, T, K), f32),
        ],
    )(x2d, wr_eff, br_eff)

    idx = jnp.transpose(idx_bt, (1, 0, 2)).reshape(T * B, K)
    wk = jnp.transpose(wk_bt, (1, 0, 2)).reshape(T * B, K)
    tokens = x2d.reshape(T * B, C, HW)

    out = pl.pallas_call(
        _expert_body,
        grid=(T * B,),
        in_specs=[
            pl.BlockSpec(memory_space=pltpu.SMEM),
            pl.BlockSpec(memory_space=pltpu.SMEM),
            pl.BlockSpec(memory_space=pltpu.SMEM),
            pl.BlockSpec((1, C, HW), lambda n: (n, 0, 0)),
            pl.BlockSpec((E, HID, C), lambda n: (0, 0, 0)),
            pl.BlockSpec((E, HID, 1), lambda n: (0, 0, 0)),
            pl.BlockSpec((E, OUT, HID), lambda n: (0, 0, 0)),
            pl.BlockSpec((E, OUT, 1), lambda n: (0, 0, 0)),
        ],
        out_specs=pl.BlockSpec((1, OUT, HW), lambda n: (n, 0, 0)),
        out_shape=jax.ShapeDtypeStruct((T * B, OUT, HW), f32),
    )(idx, wk, taus, tokens, w1_eff, b1_eff, w2_eff, b2_eff)

    return out.reshape(T, B, OUT, H, W)


# speedup vs baseline: 1.9008x; 1.0161x over previous
import jax
import jax.numpy as jnp
from jax.experimental import pallas as pl
from jax.experimental.pallas import tpu as pltpu

T, B, C, H, W = 4, 16, 256, 14, 14
HW = H * W
E, K = 8, 2
HID, OUT = 256, 256
_BN_INV = 1.0 / jnp.sqrt(jnp.float32(1.0 + 1e-5))


def _router_body(x_ref, wr_ref, br_ref, idx_ref, wk_ref):
    x = x_ref[:, 0]
    v = jnp.zeros((C, HW), jnp.float32)
    srows = []
    for t in range(T):
        v = v + (x[t] - v) / 2.0
        s = ((v - 1.0) >= 0.0).astype(jnp.float32)
        v = v * (1.0 - s)
        srows.append(jnp.sum(s, axis=-1, keepdims=True))
    S = jnp.concatenate(srows, axis=1)
    logits = jax.lax.dot_general(
        S, wr_ref[...], (((0,), (1,)), ((), ())),
        preferred_element_type=jnp.float32)
    logits = logits * (1.0 / HW) + br_ref[...]
    iota = jax.lax.broadcasted_iota(jnp.int32, (T, E), 1)
    m1 = jnp.max(logits, axis=-1, keepdims=True)
    i1 = jnp.min(jnp.where(logits >= m1, iota, E), axis=-1, keepdims=True)
    l2 = jnp.where(iota == i1, -jnp.inf, logits)
    m2 = jnp.max(l2, axis=-1, keepdims=True)
    i2 = jnp.min(jnp.where(l2 >= m2, iota, E), axis=-1, keepdims=True)
    w1 = 1.0 / (1.0 + jnp.exp(m2 - m1))
    idx_ref[0] = jnp.concatenate([i1, i2], axis=1)
    wk_ref[0] = jnp.concatenate([w1, 1.0 - w1], axis=1)


def _expert_body(idx_ref, wk_ref, taus_ref, tok_ref, w1_ref, b1_ref,
                 w2_ref, b2_ref, out_ref):
    n = pl.program_id(0)
    t, b = n // B, n % B
    tok = tok_ref[0]
    acc = jnp.zeros((OUT, HW), jnp.float32)
    for k in range(K):
        e = idx_ref[b, t, k]
        tau = taus_ref[e]
        s1 = ((tok / tau - 1.0) >= 0.0).astype(jnp.float32)
        h = jnp.dot(w1_ref[e], s1, preferred_element_type=jnp.float32) + b1_ref[e]
        x2 = tok + h
        s2 = ((x2 / tau - 1.0) >= 0.0).astype(jnp.float32)
        o = jnp.dot(w2_ref[e], s2, preferred_element_type=jnp.float32) + b2_ref[e]
        acc = acc + wk_ref[b, t, k] * (o + x2)
    out_ref[0] = acc


def kernel(x, Wr, br, gr, betar, W1, b1, g1, bt1, W2, b2, g2, bt2):
    f32 = jnp.float32
    x2d = x.reshape(T, B, C, HW)
    taus = jnp.linspace(1.5, 4.0, E).astype(f32)
    ar = gr * _BN_INV
    wr_eff = Wr * ar[:, None]
    br_eff = (br * ar + betar).reshape(1, E)
    a1 = g1 * _BN_INV
    w1_eff = W1 * a1[:, :, None]
    b1_eff = (b1 * a1 + bt1).reshape(E, HID, 1)
    a2 = g2 * _BN_INV
    w2_eff = W2 * a2[:, :, None]
    b2_eff = (b2 * a2 + bt2).reshape(E, OUT, 1)

    idx_bt, wk_bt = pl.pallas_call(
        _router_body,
        grid=(B,),
        in_specs=[
            pl.BlockSpec((T, 1, C, HW), lambda b: (0, b, 0, 0)),
            pl.BlockSpec((E, C), lambda b: (0, 0)),
            pl.BlockSpec((1, E), lambda b: (0, 0)),
        ],
        out_specs=[
            pl.BlockSpec((1, T, K), lambda b: (b, 0, 0)),
            pl.BlockSpec((1, T, K), lambda b: (b, 0, 0)),
        ],
        out_shape=[
            jax.ShapeDtypeStruct((B, T, K), jnp.int32),
            jax.ShapeDtypeStruct((B, T, K), f32),
        ],
    )(x2d, wr_eff, br_eff)

    tokens = x2d.reshape(T * B, C, HW)

    out = pl.pallas_call(
        _expert_body,
        grid=(T * B,),
        in_specs=[
            pl.BlockSpec(memory_space=pltpu.SMEM),
            pl.BlockSpec(memory_space=pltpu.SMEM),
            pl.BlockSpec(memory_space=pltpu.SMEM),
            pl.BlockSpec((1, C, HW), lambda n: (n, 0, 0)),
            pl.BlockSpec((E, HID, C), lambda n: (0, 0, 0)),
            pl.BlockSpec((E, HID, 1), lambda n: (0, 0, 0)),
            pl.BlockSpec((E, OUT, HID), lambda n: (0, 0, 0)),
            pl.BlockSpec((E, OUT, 1), lambda n: (0, 0, 0)),
        ],
        out_specs=pl.BlockSpec((1, OUT, HW), lambda n: (n, 0, 0)),
        out_shape=jax.ShapeDtypeStruct((T * B, OUT, HW), f32),
    )(idx_bt, wk_bt, taus, tokens, w1_eff, b1_eff, w2_eff, b2_eff)

    return out.reshape(T, B, OUT, H, W)


# speedup vs baseline: 1.9159x; 1.0080x over previous
import math

import jax
import jax.numpy as jnp
from jax.experimental import pallas as pl
from jax.experimental.pallas import tpu as pltpu

T, B, C, H, W = 4, 16, 256, 14, 14
HW = H * W
E, K = 8, 2
HID, OUT = 256, 256
_BN_INV = 1.0 / math.sqrt(1.0 + 1e-5)


def _router_body(x_ref, wr_ref, br_ref, idx_ref, wk_ref):
    x = x_ref[:, 0]
    v = jnp.zeros((C, HW), jnp.float32)
    srows = []
    for t in range(T):
        v = v + (x[t] - v) / 2.0
        s = ((v - 1.0) >= 0.0).astype(jnp.float32)
        v = v * (1.0 - s)
        srows.append(jnp.sum(s, axis=-1, keepdims=True))
    S = jnp.concatenate(srows, axis=1)
    logits = jax.lax.dot_general(
        S, wr_ref[...], (((0,), (1,)), ((), ())),
        preferred_element_type=jnp.float32)
    logits = logits * (1.0 / HW) + br_ref[...]
    iota = jax.lax.broadcasted_iota(jnp.int32, (T, E), 1)
    m1 = jnp.max(logits, axis=-1, keepdims=True)
    i1 = jnp.min(jnp.where(logits >= m1, iota, E), axis=-1, keepdims=True)
    l2 = jnp.where(iota == i1, -jnp.inf, logits)
    m2 = jnp.max(l2, axis=-1, keepdims=True)
    i2 = jnp.min(jnp.where(l2 >= m2, iota, E), axis=-1, keepdims=True)
    w1 = 1.0 / (1.0 + jnp.exp(m2 - m1))
    idx_ref[0] = jnp.concatenate([i1, i2], axis=1)
    wk_ref[0] = jnp.concatenate([w1, 1.0 - w1], axis=1)


def _expert_body(idx_ref, wk_ref, taus_ref, tok_ref, w1_ref, b1_ref,
                 w2_ref, b2_ref, out_ref):
    n = pl.program_id(0)
    t, b = n // B, n % B
    tok = tok_ref[0]
    acc = jnp.zeros((OUT, HW), jnp.float32)
    for k in range(K):
        e = idx_ref[b, t, k]
        tau = taus_ref[e]
        s1 = ((tok / tau - 1.0) >= 0.0).astype(jnp.bfloat16)
        h = jnp.dot(w1_ref[e], s1, preferred_element_type=jnp.float32) + b1_ref[e]
        x2 = tok + h
        s2 = ((x2 / tau - 1.0) >= 0.0).astype(jnp.bfloat16)
        o = jnp.dot(w2_ref[e], s2, preferred_element_type=jnp.float32) + b2_ref[e]
        acc = acc + wk_ref[b, t, k] * (o + x2)
    out_ref[0] = acc


def kernel(x, Wr, br, gr, betar, W1, b1, g1, bt1, W2, b2, g2, bt2):
    f32 = jnp.float32
    taus = jnp.linspace(1.5, 4.0, E).astype(f32)
    ar = gr * _BN_INV
    wr_eff = Wr * ar[:, None]
    br_eff = (br * ar + betar).reshape(1, E)
    a1 = g1 * _BN_INV
    b1_eff = (b1 * a1 + bt1).reshape(E, HID, 1)
    a2 = g2 * _BN_INV
    b2_eff = (b2 * a2 + bt2).reshape(E, OUT, 1)
    w1_bf = (W1 * a1[:, :, None]).astype(jnp.bfloat16)
    w2_bf = (W2 * a2[:, :, None]).astype(jnp.bfloat16)
    x2d = x.reshape(T, B, C, HW)

    idx_bt, wk_bt = pl.pallas_call(
        _router_body,
        grid=(B,),
        in_specs=[
            pl.BlockSpec((T, 1, C, HW), lambda b: (0, b, 0, 0)),
            pl.BlockSpec((E, C), lambda b: (0, 0)),
            pl.BlockSpec((1, E), lambda b: (0, 0)),
        ],
        out_specs=[
            pl.BlockSpec((1, T, K), lambda b: (b, 0, 0)),
            pl.BlockSpec((1, T, K), lambda b: (b, 0, 0)),
        ],
        out_shape=[
            jax.ShapeDtypeStruct((B, T, K), jnp.int32),
            jax.ShapeDtypeStruct((B, T, K), f32),
        ],
    )(x2d, wr_eff, br_eff)

    tokens = x2d.reshape(T * B, C, HW)
    out = pl.pallas_call(
        _expert_body,
        grid=(T * B,),
        in_specs=[
            pl.BlockSpec(memory_space=pltpu.SMEM),
            pl.BlockSpec(memory_space=pltpu.SMEM),
            pl.BlockSpec(memory_space=pltpu.SMEM),
            pl.BlockSpec((1, C, HW), lambda n: (n, 0, 0)),
            pl.BlockSpec((E, HID, C), lambda n: (0, 0, 0)),
            pl.BlockSpec((E, HID, 1), lambda n: (0, 0, 0)),
            pl.BlockSpec((E, OUT, HID), lambda n: (0, 0, 0)),
            pl.BlockSpec((E, OUT, 1), lambda n: (0, 0, 0)),
        ],
        out_specs=pl.BlockSpec((1, OUT, HW), lambda n: (n, 0, 0)),
        out_shape=jax.ShapeDtypeStruct((T * B, OUT, HW), f32),
    )(idx_bt, wk_bt, taus, tokens, w1_bf, b1_eff, w2_bf, b2_eff)

    return out.reshape(T, B, OUT, H, W)


# speedup vs baseline: 2.5922x; 1.3530x over previous
import math

import jax
import jax.numpy as jnp
from jax.experimental import pallas as pl
from jax.experimental.pallas import tpu as pltpu

T, B, C, H, W = 4, 16, 256, 14, 14
HW = H * W
E, K = 8, 2
HID, OUT = 256, 256
_BN_INV = 1.0 / math.sqrt(1.0 + 1e-5)


def _router_body(x_ref, wr_ref, br_ref, idx_ref, wk_ref):
    x = x_ref[:, 0]
    v = jnp.zeros((C, HW), jnp.float32)
    srows = []
    for t in range(T):
        v = v + (x[t] - v) / 2.0
        s = ((v - 1.0) >= 0.0).astype(jnp.float32)
        v = v * (1.0 - s)
        srows.append(jnp.sum(s, axis=-1, keepdims=True))
    S = jnp.concatenate(srows, axis=1)
    logits = jax.lax.dot_general(
        S, wr_ref[...], (((0,), (1,)), ((), ())),
        preferred_element_type=jnp.float32)
    logits = logits * (1.0 / HW) + br_ref[...]
    iota = jax.lax.broadcasted_iota(jnp.int32, (T, E), 1)
    m1 = jnp.max(logits, axis=-1, keepdims=True)
    i1 = jnp.min(jnp.where(logits >= m1, iota, E), axis=-1, keepdims=True)
    l2 = jnp.where(iota == i1, -jnp.inf, logits)
    m2 = jnp.max(l2, axis=-1, keepdims=True)
    i2 = jnp.min(jnp.where(l2 >= m2, iota, E), axis=-1, keepdims=True)
    w1 = 1.0 / (1.0 + jnp.exp(m2 - m1))
    idx_ref[0] = jnp.concatenate([i1, i2], axis=1)
    wk_ref[0] = jnp.concatenate([w1, 1.0 - w1], axis=1)


def _expert_body(idx_ref, wk_ref, taus_ref, tok_ref, w1_ref, b1_ref,
                 w2_ref, b2_ref, out_ref):
    n = pl.program_id(0)
    t, b = n // B, n % B
    tok = tok_ref[0, 0]
    acc = jnp.zeros((OUT, HW), jnp.float32)
    for k in range(K):
        e = idx_ref[b, t, k]
        tau = taus_ref[e]
        s1 = ((tok / tau - 1.0) >= 0.0).astype(jnp.bfloat16)
        h = jnp.dot(w1_ref[e], s1, preferred_element_type=jnp.float32) + b1_ref[e]
        x2 = tok + h
        s2 = ((x2 / tau - 1.0) >= 0.0).astype(jnp.bfloat16)
        o = jnp.dot(w2_ref[e], s2, preferred_element_type=jnp.float32) + b2_ref[e]
        acc = acc + wk_ref[b, t, k] * (o + x2)
    out_ref[0, 0] = acc


def kernel(x, Wr, br, gr, betar, W1, b1, g1, bt1, W2, b2, g2, bt2):
    f32 = jnp.float32
    taus = jnp.linspace(1.5, 4.0, E).astype(f32)
    ar = gr * _BN_INV
    wr_eff = Wr * ar[:, None]
    br_eff = (br * ar + betar).reshape(1, E)
    a1 = g1 * _BN_INV
    b1_eff = (b1 * a1 + bt1).reshape(E, HID, 1)
    a2 = g2 * _BN_INV
    b2_eff = (b2 * a2 + bt2).reshape(E, OUT, 1)
    w1_bf = (W1 * a1[:, :, None]).astype(jnp.bfloat16)
    w2_bf = (W2 * a2[:, :, None]).astype(jnp.bfloat16)
    x2d = x.reshape(T, B, C, HW)

    idx_bt, wk_bt = pl.pallas_call(
        _router_body,
        grid=(B,),
        in_specs=[
            pl.BlockSpec((T, 1, C, HW), lambda b: (0, b, 0, 0)),
            pl.BlockSpec((E, C), lambda b: (0, 0)),
            pl.BlockSpec((1, E), lambda b: (0, 0)),
        ],
        out_specs=[
            pl.BlockSpec((1, T, K), lambda b: (b, 0, 0)),
            pl.BlockSpec((1, T, K), lambda b: (b, 0, 0)),
        ],
        out_shape=[
            jax.ShapeDtypeStruct((B, T, K), jnp.int32),
            jax.ShapeDtypeStruct((B, T, K), f32),
        ],
    )(x2d, wr_eff, br_eff)

    out = pl.pallas_call(
        _expert_body,
        grid=(T * B,),
        in_specs=[
            pl.BlockSpec(memory_space=pltpu.SMEM),
            pl.BlockSpec(memory_space=pltpu.SMEM),
            pl.BlockSpec(memory_space=pltpu.SMEM),
            pl.BlockSpec((1, 1, C, HW), lambda n: (n // B, n % B, 0, 0)),
            pl.BlockSpec((E, HID, C), lambda n: (0, 0, 0)),
            pl.BlockSpec((E, HID, 1), lambda n: (0, 0, 0)),
            pl.BlockSpec((E, OUT, HID), lambda n: (0, 0, 0)),
            pl.BlockSpec((E, OUT, 1), lambda n: (0, 0, 0)),
        ],
        out_specs=pl.BlockSpec((1, 1, OUT, HW),
                               lambda n: (n // B, n % B, 0, 0)),
        out_shape=jax.ShapeDtypeStruct((T, B, OUT, HW), f32),
    )(idx_bt, wk_bt, taus, x2d, w1_bf, b1_eff, w2_bf, b2_eff)

    return out.reshape(T, B, OUT, H, W)


# speedup vs baseline: 3.3930x; 1.3089x over previous
import math

import jax
import jax.numpy as jnp
from jax.experimental import pallas as pl
from jax.experimental.pallas import tpu as pltpu

T, B, C, H, W = 4, 16, 256, 14, 14
HW = H * W
E, K = 8, 2
HID, OUT = 256, 256
_BN_INV = 1.0 / math.sqrt(1.0 + 1e-5)


def _router_body(x_ref, wr_ref, br_ref, idx_ref, wk_ref):
    x = x_ref[:, 0]
    v = jnp.zeros((C, HW), jnp.float32)
    srows = []
    for t in range(T):
        v = v + (x[t] - v) / 2.0
        s = ((v - 1.0) >= 0.0).astype(jnp.float32)
        v = v * (1.0 - s)
        srows.append(jnp.sum(s, axis=-1, keepdims=True))
    S = jnp.concatenate(srows, axis=1)
    logits = jax.lax.dot_general(
        S, wr_ref[...], (((0,), (1,)), ((), ())),
        preferred_element_type=jnp.float32)
    logits = logits * (1.0 / HW) + br_ref[...]
    iota = jax.lax.broadcasted_iota(jnp.int32, (T, E), 1)
    m1 = jnp.max(logits, axis=-1, keepdims=True)
    i1 = jnp.min(jnp.where(logits >= m1, iota, E), axis=-1, keepdims=True)
    l2 = jnp.where(iota == i1, -jnp.inf, logits)
    m2 = jnp.max(l2, axis=-1, keepdims=True)
    i2 = jnp.min(jnp.where(l2 >= m2, iota, E), axis=-1, keepdims=True)
    w1 = 1.0 / (1.0 + jnp.exp(m2 - m1))
    idx_ref[0] = jnp.concatenate([i1, i2], axis=1)
    wk_ref[0] = jnp.concatenate([w1, 1.0 - w1], axis=1)


def _expert_body(idx_ref, wk_ref, taus_ref, tok_ref, w1_ref, b1_ref,
                 w2_ref, b2_ref, out_ref):
    b = pl.program_id(0)
    for t in range(T):
        tok = tok_ref[t, 0]
        acc = jnp.zeros((OUT, HW), jnp.float32)
        for k in range(K):
            e = idx_ref[b, t, k]
            tau = taus_ref[e]
            s1 = ((tok / tau - 1.0) >= 0.0).astype(jnp.bfloat16)
            h = jnp.dot(w1_ref[e], s1,
                        preferred_element_type=jnp.float32) + b1_ref[e]
            x2 = tok + h
            s2 = ((x2 / tau - 1.0) >= 0.0).astype(jnp.bfloat16)
            o = jnp.dot(w2_ref[e], s2,
                        preferred_element_type=jnp.float32) + b2_ref[e]
            acc = acc + wk_ref[b, t, k] * (o + x2)
        out_ref[t, 0] = acc


def kernel(x, Wr, br, gr, betar, W1, b1, g1, bt1, W2, b2, g2, bt2):
    f32 = jnp.float32
    taus = jnp.linspace(1.5, 4.0, E).astype(f32)
    ar = gr * _BN_INV
    wr_eff = Wr * ar[:, None]
    br_eff = (br * ar + betar).reshape(1, E)
    a1 = g1 * _BN_INV
    b1_eff = (b1 * a1 + bt1).reshape(E, HID, 1)
    a2 = g2 * _BN_INV
    b2_eff = (b2 * a2 + bt2).reshape(E, OUT, 1)
    w1_bf = (W1 * a1[:, :, None]).astype(jnp.bfloat16)
    w2_bf = (W2 * a2[:, :, None]).astype(jnp.bfloat16)
    x2d = x.reshape(T, B, C, HW)

    idx_bt, wk_bt = pl.pallas_call(
        _router_body,
        grid=(B,),
        in_specs=[
            pl.BlockSpec((T, 1, C, HW), lambda b: (0, b, 0, 0)),
            pl.BlockSpec((E, C), lambda b: (0, 0)),
            pl.BlockSpec((1, E), lambda b: (0, 0)),
        ],
        out_specs=[
            pl.BlockSpec((1, T, K), lambda b: (b, 0, 0)),
            pl.BlockSpec((1, T, K), lambda b: (b, 0, 0)),
        ],
        out_shape=[
            jax.ShapeDtypeStruct((B, T, K), jnp.int32),
            jax.ShapeDtypeStruct((B, T, K), f32),
        ],
    )(x2d, wr_eff, br_eff)

    out = pl.pallas_call(
        _expert_body,
        grid=(B,),
        in_specs=[
            pl.BlockSpec(memory_space=pltpu.SMEM),
            pl.BlockSpec(memory_space=pltpu.SMEM),
            pl.BlockSpec(memory_space=pltpu.SMEM),
            pl.BlockSpec((T, 1, C, HW), lambda b: (0, b, 0, 0)),
            pl.BlockSpec((E, HID, C), lambda b: (0, 0, 0)),
            pl.BlockSpec((E, HID, 1), lambda b: (0, 0, 0)),
            pl.BlockSpec((E, OUT, HID), lambda b: (0, 0, 0)),
            pl.BlockSpec((E, OUT, 1), lambda b: (0, 0, 0)),
        ],
        out_specs=pl.BlockSpec((T, 1, OUT, HW), lambda b: (0, b, 0, 0)),
        out_shape=jax.ShapeDtypeStruct((T, B, OUT, HW), f32),
    )(idx_bt, wk_bt, taus, x2d, w1_bf, b1_eff, w2_bf, b2_eff)

    return out.reshape(T, B, OUT, H, W)


# speedup vs baseline: 3.6141x; 1.0652x over previous
import math

import jax
import jax.numpy as jnp
from jax.experimental import pallas as pl
from jax.experimental.pallas import tpu as pltpu

T, B, C, H, W = 4, 16, 256, 14, 14
HW = H * W
E, K = 8, 2
HID, OUT = 256, 256
_BN_INV = 1.0 / math.sqrt(1.0 + 1e-5)


def _fused_body(taus_ref, x_ref, wr_ref, br_ref, w1_ref, b1_ref,
                w2_ref, b2_ref, out_ref, idx_s, wk_s):
    x = x_ref[:, 0]
    v = jnp.zeros((C, HW), jnp.float32)
    srows = []
    for t in range(T):
        v = v + (x[t] - v) / 2.0
        s = ((v - 1.0) >= 0.0).astype(jnp.float32)
        v = v * (1.0 - s)
        srows.append(jnp.sum(s, axis=-1, keepdims=True))
    S = jnp.concatenate(srows, axis=1)
    logits = jax.lax.dot_general(
        S, wr_ref[...], (((0,), (1,)), ((), ())),
        preferred_element_type=jnp.float32)
    logits = logits * (1.0 / HW) + br_ref[...]
    iota = jax.lax.broadcasted_iota(jnp.int32, (T, E), 1)
    m1 = jnp.max(logits, axis=-1, keepdims=True)
    i1 = jnp.min(jnp.where(logits >= m1, iota, E), axis=-1, keepdims=True)
    l2 = jnp.where(iota == i1, -jnp.inf, logits)
    m2 = jnp.max(l2, axis=-1, keepdims=True)
    i2 = jnp.min(jnp.where(l2 >= m2, iota, E), axis=-1, keepdims=True)
    w1 = 1.0 / (1.0 + jnp.exp(m2 - m1))
    idx_s[...] = jnp.concatenate([i1, i2], axis=1)
    wk_s[...] = jnp.concatenate([w1, 1.0 - w1], axis=1)
    for t in range(T):
        tok = x[t]
        acc = jnp.zeros((OUT, HW), jnp.float32)
        for k in range(K):
            e = idx_s[t, k]
            tau = taus_ref[e]
            s1 = ((tok / tau - 1.0) >= 0.0).astype(jnp.bfloat16)
            h = jnp.dot(w1_ref[e], s1,
                        preferred_element_type=jnp.float32) + b1_ref[e]
            x2 = tok + h
            s2 = ((x2 / tau - 1.0) >= 0.0).astype(jnp.bfloat16)
            o = jnp.dot(w2_ref[e], s2,
                        preferred_element_type=jnp.float32) + b2_ref[e]
            acc = acc + wk_s[t, k] * (o + x2)
        out_ref[t, 0] = acc


def kernel(x, Wr, br, gr, betar, W1, b1, g1, bt1, W2, b2, g2, bt2):
    f32 = jnp.float32
    taus = jnp.linspace(1.5, 4.0, E).astype(f32)
    ar = gr * _BN_INV
    wr_eff = Wr * ar[:, None]
    br_eff = (br * ar + betar).reshape(1, E)
    a1 = g1 * _BN_INV
    b1_eff = (b1 * a1 + bt1).reshape(E, HID, 1)
    a2 = g2 * _BN_INV
    b2_eff = (b2 * a2 + bt2).reshape(E, OUT, 1)
    w1_bf = (W1 * a1[:, :, None]).astype(jnp.bfloat16)
    w2_bf = (W2 * a2[:, :, None]).astype(jnp.bfloat16)
    x2d = x.reshape(T, B, C, HW)

    out = pl.pallas_call(
        _fused_body,
        grid=(B,),
        in_specs=[
            pl.BlockSpec(memory_space=pltpu.SMEM),
            pl.BlockSpec((T, 1, C, HW), lambda b: (0, b, 0, 0)),
            pl.BlockSpec((E, C), lambda b: (0, 0)),
            pl.BlockSpec((1, E), lambda b: (0, 0)),
            pl.BlockSpec((E, HID, C), lambda b: (0, 0, 0)),
            pl.BlockSpec((E, HID, 1), lambda b: (0, 0, 0)),
            pl.BlockSpec((E, OUT, HID), lambda b: (0, 0, 0)),
            pl.BlockSpec((E, OUT, 1), lambda b: (0, 0, 0)),
        ],
        out_specs=pl.BlockSpec((T, 1, OUT, HW), lambda b: (0, b, 0, 0)),
        out_shape=jax.ShapeDtypeStruct((T, B, OUT, HW), f32),
        scratch_shapes=[
            pltpu.VMEM((T, K), jnp.int32),
            pltpu.VMEM((T, K), f32),
        ],
    )(taus, x2d, wr_eff, br_eff, w1_bf, b1_eff, w2_bf, b2_eff)

    return out.reshape(T, B, OUT, H, W)
